# Initial kernel scaffold; baseline (speedup 1.0000x reference)
import jax
import jax.numpy as jnp
from jax.experimental import pallas as pl


def kernel(x, edge_index, W1l, b1, W1r, W2l, b2, W2r, Wh, bh):
    raise NotImplementedError("write your pallas kernel here")


# speedup vs baseline: 4.6684x; 4.6684x over previous
import functools

import jax
import jax.numpy as jnp
from jax import lax
from jax.experimental import pallas as pl
from jax.experimental.pallas import tpu as pltpu
from jax.experimental.pallas import tpu_sc as plsc

_NC = 2
_NS = 16
_NW = _NC * _NS


@functools.lru_cache(maxsize=None)
def _make_sc_agg(n, e, d, with_cnt, k=80):
    assert e % _NW == 0
    epw = e // _NW
    assert epw % k == 0
    iters = epw // k
    n_pad = -(-n // (128 * _NS)) * (128 * _NS)
    rpw = n_pad // _NS

    mesh = plsc.VectorSubcoreMesh(core_axis_name="c", subcore_axis_name="s",
                                  num_cores=_NC, num_subcores=_NS)
    out_type = [jax.ShapeDtypeStruct((_NC, n_pad, d), jnp.float32)]
    scratch = [
        pltpu.VMEM((k,), jnp.int32),
        pltpu.VMEM((k,), jnp.int32),
        pltpu.VMEM((k, d), jnp.float32),
        pltpu.VMEM_SHARED((n_pad, d), jnp.float32),
        pltpu.SemaphoreType.DMA,
    ]
    if with_cnt:
        out_type.append(
            jax.ShapeDtypeStruct((_NC, n_pad, d), jnp.float32))
        scratch += [
            pltpu.VMEM((k, d), jnp.float32),
        ]

    def body_cnt(x_hbm, src_hbm, dst_hbm, zacc_hbm, ones_hbm,
                 acc_out, cnt_out,
                 src_v, dst_v, rows_v, acc_sh, sem, ones_v):
        c = lax.axis_index("c")
        s = lax.axis_index("s")
        roff = s * rpw
        pltpu.sync_copy(zacc_hbm.at[pl.ds(roff, rpw)],
                        acc_sh.at[pl.ds(roff, rpw)])
        pltpu.sync_copy(ones_hbm, ones_v)
        plsc.subcore_barrier()

        ebase = (c * _NS + s) * epw

        def step(i, carry):
            base = ebase + i * k
            pltpu.sync_copy(src_hbm.at[pl.ds(base, k)], src_v)
            pltpu.sync_copy(dst_hbm.at[pl.ds(base, k)], dst_v)
            pltpu.async_copy(x_hbm.at[src_v], rows_v, sem).wait()
            pltpu.sync_copy(rows_v, acc_sh.at[dst_v], add=True)
            return carry
        lax.fori_loop(0, iters, step, 0)

        plsc.subcore_barrier()
        pltpu.sync_copy(acc_sh.at[pl.ds(roff, rpw)],
                        acc_out.at[c, pl.ds(roff, rpw)])
        plsc.subcore_barrier()

        pltpu.sync_copy(zacc_hbm.at[pl.ds(roff, rpw)],
                        acc_sh.at[pl.ds(roff, rpw)])
        plsc.subcore_barrier()

        def step2(i, carry):
            base = ebase + i * k
            pltpu.sync_copy(dst_hbm.at[pl.ds(base, k)], dst_v)
            pltpu.sync_copy(ones_v, acc_sh.at[dst_v], add=True)
            return carry
        lax.fori_loop(0, iters, step2, 0)

        plsc.subcore_barrier()
        pltpu.sync_copy(acc_sh.at[pl.ds(roff, rpw)],
                        cnt_out.at[c, pl.ds(roff, rpw)])

    def body_plain(x_hbm, src_hbm, dst_hbm, zacc_hbm,
                   acc_out,
                   src_v, dst_v, rows_v, acc_sh, sem):
        c = lax.axis_index("c")
        s = lax.axis_index("s")
        roff = s * rpw
        pltpu.sync_copy(zacc_hbm.at[pl.ds(roff, rpw)],
                        acc_sh.at[pl.ds(roff, rpw)])
        plsc.subcore_barrier()

        ebase = (c * _NS + s) * epw

        def step(i, carry):
            base = ebase + i * k
            pltpu.sync_copy(src_hbm.at[pl.ds(base, k)], src_v)
            pltpu.sync_copy(dst_hbm.at[pl.ds(base, k)], dst_v)
            pltpu.async_copy(x_hbm.at[src_v], rows_v, sem).wait()
            pltpu.sync_copy(rows_v, acc_sh.at[dst_v], add=True)
            return carry
        lax.fori_loop(0, iters, step, 0)

        plsc.subcore_barrier()
        pltpu.sync_copy(acc_sh.at[pl.ds(roff, rpw)],
                        acc_out.at[c, pl.ds(roff, rpw)])

    return pl.kernel(body_cnt if with_cnt else body_plain,
                     out_type=tuple(out_type) if with_cnt else out_type[0],
                     mesh=mesh, scratch_types=scratch)


def _sage_dense(acc, cnt, x, Wl, Wr, b, blk=1000):
    n, d = x.shape
    h = Wl.shape[1]
    assert n % blk == 0

    def kern(acc_ref, cnt_ref, x_ref, wl_ref, wr_ref, b_ref, o_ref):
        a = acc_ref[0] + acc_ref[1]
        c = cnt_ref[0] + cnt_ref[1]
        m = a / jnp.maximum(c, 1.0)
        y = (jnp.dot(m, wl_ref[...], preferred_element_type=jnp.float32,
                     precision=lax.Precision.HIGHEST)
             + jnp.dot(x_ref[...], wr_ref[...],
                       preferred_element_type=jnp.float32,
                       precision=lax.Precision.HIGHEST)
             + b_ref[...])
        o_ref[...] = jnp.maximum(y, 0.0)

    return pl.pallas_call(
        kern,
        grid=(n // blk,),
        in_specs=[
            pl.BlockSpec((_NC, blk, d), lambda i: (0, i, 0)),
            pl.BlockSpec((_NC, blk, 1), lambda i: (0, i, 0)),
            pl.BlockSpec((blk, d), lambda i: (i, 0)),
            pl.BlockSpec((d, h), lambda i: (0, 0)),
            pl.BlockSpec((d, h), lambda i: (0, 0)),
            pl.BlockSpec((1, h), lambda i: (0, 0)),
        ],
        out_specs=pl.BlockSpec((blk, h), lambda i: (i, 0)),
        out_shape=jax.ShapeDtypeStruct((n, h), jnp.float32),
    )(acc, cnt, x, Wl, Wr, b)


def _sage_dense_head(acc, cnt, x, Wl, Wr, b, whT, bh, blk=1000):
    n, d = x.shape
    h = Wl.shape[1]
    assert n % blk == 0

    def kern(acc_ref, cnt_ref, x_ref, wl_ref, wr_ref, b_ref, wh_ref, bh_ref,
             o_ref):
        a = acc_ref[0] + acc_ref[1]
        c = cnt_ref[0] + cnt_ref[1]
        m = a / jnp.maximum(c, 1.0)
        y = (jnp.dot(m, wl_ref[...], preferred_element_type=jnp.float32,
                     precision=lax.Precision.HIGHEST)
             + jnp.dot(x_ref[...], wr_ref[...],
                       preferred_element_type=jnp.float32,
                       precision=lax.Precision.HIGHEST)
             + b_ref[...])
        h2 = jnp.maximum(y, 0.0)
        o_ref[...] = (jnp.sum(h2 * wh_ref[...], axis=1, keepdims=True)
                      + bh_ref[0:1, 0:1])

    return pl.pallas_call(
        kern,
        grid=(n // blk,),
        in_specs=[
            pl.BlockSpec((_NC, blk, d), lambda i: (0, i, 0)),
            pl.BlockSpec((_NC, blk, 1), lambda i: (0, i, 0)),
            pl.BlockSpec((blk, d), lambda i: (i, 0)),
            pl.BlockSpec((d, h), lambda i: (0, 0)),
            pl.BlockSpec((d, h), lambda i: (0, 0)),
            pl.BlockSpec((1, h), lambda i: (0, 0)),
            pl.BlockSpec((1, h), lambda i: (0, 0)),
            pl.BlockSpec((1, 1), lambda i: (0, 0)),
        ],
        out_specs=pl.BlockSpec((blk, 1), lambda i: (i, 0)),
        out_shape=jax.ShapeDtypeStruct((n, 1), jnp.float32),
    )(acc, cnt, x, Wl, Wr, b, whT, bh)


def kernel(x, edge_index, W1l, b1, W1r, W2l, b2, W2r, Wh, bh):
    n, d = x.shape
    e = edge_index.shape[1]
    h = W1l.shape[1]

    src = edge_index[0]
    dst = edge_index[1]
    n_pad = -(-n // (128 * _NS)) * (128 * _NS)
    zacc = jnp.zeros((n_pad, d), jnp.float32)
    ones = jnp.ones((80, d), jnp.float32)

    acc1, cntp = _make_sc_agg(n, e, d, True)(x, src, dst, zacc, ones)
    cnt = cntp[:, :, 0:1]
    h1 = _sage_dense(acc1, cnt, x, W1l, W1r, b1.reshape(1, h))
    acc2 = _make_sc_agg(n, e, h, False)(h1, src, dst, zacc)
    out = _sage_dense_head(acc2, cnt, h1, W2l, W2r, b2.reshape(1, h),
                           Wh.reshape(1, h), bh.reshape(1, 1))
    return out[:, 0]


# speedup vs baseline: 6.7360x; 1.4429x over previous
import functools

import jax
import jax.numpy as jnp
from jax import lax
from jax.experimental import pallas as pl
from jax.experimental.pallas import tpu as pltpu
from jax.experimental.pallas import tpu_sc as plsc

_NC = 2
_NS = 16
_NW = _NC * _NS


@functools.lru_cache(maxsize=None)
def _make_sc_agg(n, e, d, with_cnt, k=80):
    assert e % _NW == 0
    epw = e // _NW
    assert epw % k == 0
    iters = epw // k
    n_pad = -(-n // (128 * _NS)) * (128 * _NS)
    rpw = n_pad // _NS

    mesh = plsc.VectorSubcoreMesh(core_axis_name="c", subcore_axis_name="s",
                                  num_cores=_NC, num_subcores=_NS)
    out_type = [jax.ShapeDtypeStruct((_NC, n_pad, d), jnp.float32)]
    scratch = [
        pltpu.VMEM((2, k), jnp.int32),
        pltpu.VMEM((2, k), jnp.int32),
        pltpu.VMEM((2, k, d), jnp.float32),
        pltpu.VMEM_SHARED((n_pad, d), jnp.float32),
        pltpu.SemaphoreType.DMA,
        pltpu.SemaphoreType.DMA,
    ]
    if with_cnt:
        out_type.append(jax.ShapeDtypeStruct((_NC, n_pad, d), jnp.float32))
        scratch.append(pltpu.VMEM((k, d), jnp.float32))

    def _pipelined_gather_scatter(x_hbm, src_hbm, dst_hbm, acc_sh,
                                  src2, dst2, rows2, sems, ebase):
        def load_idx(ci, p):
            pltpu.sync_copy(src_hbm.at[pl.ds(ebase + ci * k, k)], src2.at[p])
            pltpu.sync_copy(dst_hbm.at[pl.ds(ebase + ci * k, k)], dst2.at[p])

        def gather_start(p):
            pltpu.async_copy(x_hbm.at[src2.at[p]], rows2.at[p], sems[p])

        def gather_wait(p):
            pltpu.make_async_copy(x_hbm.at[src2.at[p]], rows2.at[p],
                                  sems[p]).wait()

        def scatter(p):
            pltpu.sync_copy(rows2.at[p], acc_sh.at[dst2.at[p]], add=True)

        load_idx(0, 0)
        gather_start(0)

        def body(t, carry):
            a = 2 * t
            b = 2 * t + 1

            @pl.when(b < iters)
            def _():
                load_idx(b, 1)
                gather_start(1)

            gather_wait(0)
            scatter(0)

            @pl.when(a + 2 < iters)
            def _():
                load_idx(a + 2, 0)
                gather_start(0)

            @pl.when(b < iters)
            def _():
                gather_wait(1)
                scatter(1)
            return carry

        lax.fori_loop(0, (iters + 1) // 2, body, 0)

    def body_cnt(x_hbm, src_hbm, dst_hbm, zacc_hbm, ones_hbm,
                 acc_out, cnt_out,
                 src2, dst2, rows2, acc_sh, sem0, sem1, ones_v):
        c = lax.axis_index("c")
        s = lax.axis_index("s")
        roff = s * rpw
        pltpu.sync_copy(zacc_hbm.at[pl.ds(roff, rpw)],
                        acc_sh.at[pl.ds(roff, rpw)])
        pltpu.sync_copy(ones_hbm, ones_v)
        plsc.subcore_barrier()

        ebase = (c * _NS + s) * epw
        _pipelined_gather_scatter(x_hbm, src_hbm, dst_hbm, acc_sh,
                                  src2, dst2, rows2, (sem0, sem1), ebase)

        plsc.subcore_barrier()
        pltpu.sync_copy(acc_sh.at[pl.ds(roff, rpw)],
                        acc_out.at[c, pl.ds(roff, rpw)])
        plsc.subcore_barrier()

        pltpu.sync_copy(zacc_hbm.at[pl.ds(roff, rpw)],
                        acc_sh.at[pl.ds(roff, rpw)])
        plsc.subcore_barrier()

        def step2(i, carry):
            pltpu.sync_copy(dst_hbm.at[pl.ds(ebase + i * k, k)], dst2.at[0])
            pltpu.sync_copy(ones_v, acc_sh.at[dst2.at[0]], add=True)
            return carry
        lax.fori_loop(0, iters, step2, 0)

        plsc.subcore_barrier()
        pltpu.sync_copy(acc_sh.at[pl.ds(roff, rpw)],
                        cnt_out.at[c, pl.ds(roff, rpw)])

    def body_plain(x_hbm, src_hbm, dst_hbm, zacc_hbm,
                   acc_out,
                   src2, dst2, rows2, acc_sh, sem0, sem1):
        c = lax.axis_index("c")
        s = lax.axis_index("s")
        roff = s * rpw
        pltpu.sync_copy(zacc_hbm.at[pl.ds(roff, rpw)],
                        acc_sh.at[pl.ds(roff, rpw)])
        plsc.subcore_barrier()

        ebase = (c * _NS + s) * epw
        _pipelined_gather_scatter(x_hbm, src_hbm, dst_hbm, acc_sh,
                                  src2, dst2, rows2, (sem0, sem1), ebase)

        plsc.subcore_barrier()
        pltpu.sync_copy(acc_sh.at[pl.ds(roff, rpw)],
                        acc_out.at[c, pl.ds(roff, rpw)])

    return pl.kernel(body_cnt if with_cnt else body_plain,
                     out_type=tuple(out_type) if with_cnt else out_type[0],
                     mesh=mesh, scratch_types=scratch)


def _sage_dense(acc, cnt, x, Wl, Wr, b, blk=1000):
    n, d = x.shape
    h = Wl.shape[1]
    assert n % blk == 0

    def kern(acc_ref, cnt_ref, x_ref, wl_ref, wr_ref, b_ref, o_ref):
        a = acc_ref[0] + acc_ref[1]
        c = cnt_ref[0] + cnt_ref[1]
        m = a / jnp.maximum(c, 1.0)
        y = (jnp.dot(m, wl_ref[...], preferred_element_type=jnp.float32,
                     precision=lax.Precision.HIGHEST)
             + jnp.dot(x_ref[...], wr_ref[...],
                       preferred_element_type=jnp.float32,
                       precision=lax.Precision.HIGHEST)
             + b_ref[...])
        o_ref[...] = jnp.maximum(y, 0.0)

    return pl.pallas_call(
        kern,
        grid=(n // blk,),
        in_specs=[
            pl.BlockSpec((_NC, blk, d), lambda i: (0, i, 0)),
            pl.BlockSpec((_NC, blk, 1), lambda i: (0, i, 0)),
            pl.BlockSpec((blk, d), lambda i: (i, 0)),
            pl.BlockSpec((d, h), lambda i: (0, 0)),
            pl.BlockSpec((d, h), lambda i: (0, 0)),
            pl.BlockSpec((1, h), lambda i: (0, 0)),
        ],
        out_specs=pl.BlockSpec((blk, h), lambda i: (i, 0)),
        out_shape=jax.ShapeDtypeStruct((n, h), jnp.float32),
    )(acc, cnt, x, Wl, Wr, b)


def _sage_dense_head(acc, cnt, x, Wl, Wr, b, whT, bh, blk=1000):
    n, d = x.shape
    h = Wl.shape[1]
    assert n % blk == 0

    def kern(acc_ref, cnt_ref, x_ref, wl_ref, wr_ref, b_ref, wh_ref, bh_ref,
             o_ref):
        a = acc_ref[0] + acc_ref[1]
        c = cnt_ref[0] + cnt_ref[1]
        m = a / jnp.maximum(c, 1.0)
        y = (jnp.dot(m, wl_ref[...], preferred_element_type=jnp.float32,
                     precision=lax.Precision.HIGHEST)
             + jnp.dot(x_ref[...], wr_ref[...],
                       preferred_element_type=jnp.float32,
                       precision=lax.Precision.HIGHEST)
             + b_ref[...])
        h2 = jnp.maximum(y, 0.0)
        o_ref[...] = (jnp.sum(h2 * wh_ref[...], axis=1, keepdims=True)
                      + bh_ref[0:1, 0:1])

    return pl.pallas_call(
        kern,
        grid=(n // blk,),
        in_specs=[
            pl.BlockSpec((_NC, blk, d), lambda i: (0, i, 0)),
            pl.BlockSpec((_NC, blk, 1), lambda i: (0, i, 0)),
            pl.BlockSpec((blk, d), lambda i: (i, 0)),
            pl.BlockSpec((d, h), lambda i: (0, 0)),
            pl.BlockSpec((d, h), lambda i: (0, 0)),
            pl.BlockSpec((1, h), lambda i: (0, 0)),
            pl.BlockSpec((1, h), lambda i: (0, 0)),
            pl.BlockSpec((1, 1), lambda i: (0, 0)),
        ],
        out_specs=pl.BlockSpec((blk, 1), lambda i: (i, 0)),
        out_shape=jax.ShapeDtypeStruct((n, 1), jnp.float32),
    )(acc, cnt, x, Wl, Wr, b, whT, bh)


def kernel(x, edge_index, W1l, b1, W1r, W2l, b2, W2r, Wh, bh):
    n, d = x.shape
    e = edge_index.shape[1]
    h = W1l.shape[1]

    src = edge_index[0]
    dst = edge_index[1]
    n_pad = -(-n // (128 * _NS)) * (128 * _NS)
    zacc = jnp.zeros((n_pad, d), jnp.float32)
    ones = jnp.ones((80, d), jnp.float32)

    acc1, cntp = _make_sc_agg(n, e, d, True)(x, src, dst, zacc, ones)
    cnt = cntp[:, :, 0:1]
    h1 = _sage_dense(acc1, cnt, x, W1l, W1r, b1.reshape(1, h))
    acc2 = _make_sc_agg(n, e, h, False)(h1, src, dst, zacc)
    out = _sage_dense_head(acc2, cnt, h1, W2l, W2r, b2.reshape(1, h),
                           Wh.reshape(1, h), bh.reshape(1, 1))
    return out[:, 0]
